# baseline (speedup 1.0000x reference)
import functools

import jax
import jax.numpy as jnp
from jax import lax
from jax.experimental import pallas as pl
from jax.experimental.pallas import tpu as pltpu
from jax.experimental.pallas import tpu_sc as plsc

NFEAT = 1000000
NEMB = 16
BATCH = 16384
NFIELDS = 26
TOT = BATCH * NFIELDS
NC, NS, NLANE = 2, 16, 16
NW = NC * NS
B_PER_W = TOT // NW
FIRE_SZ = 128
FIRES = 13
CHUNK = FIRES * FIRE_SZ
NCHUNK = B_PER_W // CHUNK

_mesh = plsc.VectorSubcoreMesh(core_axis_name="c", subcore_axis_name="s")


@functools.partial(
    pl.kernel,
    mesh=_mesh,
    compiler_params=pltpu.CompilerParams(use_tc_tiling_on_sc=False),
    out_type=jax.ShapeDtypeStruct((NW, NCHUNK, CHUNK, NEMB), jnp.float32),
    scratch_types=[
        pltpu.VMEM((FIRES, FIRE_SZ), jnp.int32),
        pltpu.VMEM((CHUNK,), jnp.float32),
        pltpu.VMEM((CHUNK, NEMB), jnp.float32),
        pltpu.SemaphoreType.DMA,
    ],
)
def _emb_lookup(w_hbm, idx_hbm, val_hbm, out_hbm, idx_v, val_v, rows_v, sem):
    wid = lax.axis_index("s") * NC + lax.axis_index("c")

    def chunk_body(c, carry):
        pltpu.sync_copy(idx_hbm.at[wid, c], idx_v)
        pltpu.sync_copy(val_hbm.at[wid, c], val_v)
        copies = []
        for j in range(FIRES):
            copies.append(
                pltpu.async_copy(
                    w_hbm.at[idx_v.at[j]],
                    rows_v.at[pl.ds(j * FIRE_SZ, FIRE_SZ)],
                    sem,
                )
            )
        for cp in copies:
            cp.wait()

        def grp_body(g, carry2):
            base = g * NLANE
            vv = val_v[pl.ds(base, NLANE)]
            for j in range(NLANE):
                rows_v[base + j, :] = rows_v[base + j, :] * vv[j]
            return carry2

        lax.fori_loop(0, CHUNK // NLANE, grp_body, 0)
        pltpu.sync_copy(rows_v, out_hbm.at[wid, c])
        return carry

    lax.fori_loop(0, NCHUNK, chunk_body, 0)


def kernel(id, value, W):
    ids = id.astype(jnp.int32).reshape(NW, NCHUNK, FIRES, FIRE_SZ)
    vals = value.reshape(NW, NCHUNK, CHUNK)
    out = _emb_lookup(W, ids, vals)
    return out.reshape(BATCH, NFIELDS, NEMB)


# speedup vs baseline: 1.0169x; 1.0169x over previous
import functools

import jax
import jax.numpy as jnp
from jax import lax
from jax.experimental import pallas as pl
from jax.experimental.pallas import tpu as pltpu
from jax.experimental.pallas import tpu_sc as plsc

NFEAT = 1000000
NEMB = 16
BATCH = 16384
NFIELDS = 26
TOT = BATCH * NFIELDS
NC, NS, NLANE = 2, 16, 16
NW = NC * NS
B_PER_W = TOT // NW
FIRE_SZ = 128
FIRES = 13
CHUNK = FIRES * FIRE_SZ
NCHUNK = B_PER_W // CHUNK

_mesh = plsc.VectorSubcoreMesh(core_axis_name="c", subcore_axis_name="s")


@functools.partial(
    pl.kernel,
    mesh=_mesh,
    compiler_params=pltpu.CompilerParams(use_tc_tiling_on_sc=False),
    out_type=jax.ShapeDtypeStruct((NW, NCHUNK, CHUNK, NEMB), jnp.float32),
    scratch_types=[
        pltpu.VMEM((2, FIRES, FIRE_SZ), jnp.int32),
        pltpu.VMEM((2, CHUNK), jnp.float32),
        pltpu.VMEM((2, CHUNK, NEMB), jnp.float32),
        pltpu.SemaphoreType.DMA,
        pltpu.SemaphoreType.DMA,
    ],
)
def _emb_lookup(w_hbm, idx_hbm, val_hbm, out_hbm, idx_v, val_v, rows_v, gsem, osem):
    wid = lax.axis_index("s") * NC + lax.axis_index("c")

    def stage(c, buf):
        pltpu.sync_copy(idx_hbm.at[wid, c], idx_v.at[buf])
        pltpu.sync_copy(val_hbm.at[wid, c], val_v.at[buf])
        return [
            pltpu.async_copy(
                w_hbm.at[idx_v.at[buf, j]],
                rows_v.at[buf, pl.ds(j * FIRE_SZ, FIRE_SZ)],
                gsem,
            )
            for j in range(FIRES)
        ]

    def scale(buf):
        def grp_body(g, carry):
            base = g * NLANE
            vv = val_v[buf, pl.ds(base, NLANE)]
            for j in range(NLANE):
                rows_v[buf, base + j, :] = rows_v[buf, base + j, :] * vv[j]
            return carry

        lax.fori_loop(0, CHUNK // NLANE, grp_body, 0)

    gathers = [None, None]
    out_copies = [None, None]
    gathers[0] = stage(0, 0)
    for c in range(NCHUNK):
        buf = c & 1
        if c + 1 < NCHUNK:
            if out_copies[1 - buf] is not None:
                out_copies[1 - buf].wait()
                out_copies[1 - buf] = None
            gathers[1 - buf] = stage(c + 1, 1 - buf)
        for cp in gathers[buf]:
            cp.wait()
        scale(buf)
        out_copies[buf] = pltpu.async_copy(
            rows_v.at[buf], out_hbm.at[wid, c], osem
        )
    for cp in out_copies:
        if cp is not None:
            cp.wait()


def kernel(id, value, W):
    ids = id.astype(jnp.int32).reshape(NW, NCHUNK, FIRES, FIRE_SZ)
    vals = value.reshape(NW, NCHUNK, CHUNK)
    out = _emb_lookup(W, ids, vals)
    return out.reshape(BATCH, NFIELDS, NEMB)


# speedup vs baseline: 1.1609x; 1.1416x over previous
import functools

import jax
import jax.numpy as jnp
from jax import lax
from jax.experimental import pallas as pl
from jax.experimental.pallas import tpu as pltpu
from jax.experimental.pallas import tpu_sc as plsc

NFEAT = 1000000
NEMB = 16
BATCH = 16384
NFIELDS = 26
NC, NS, NLANE = 2, 16, 16
NW = NC * NS
ROWS_PER_W = BATCH // NW
RCHUNK = 64
NCHUNK = ROWS_PER_W // RCHUNK

_mesh = plsc.VectorSubcoreMesh(core_axis_name="c", subcore_axis_name="s")


@functools.partial(
    pl.kernel,
    mesh=_mesh,
    compiler_params=pltpu.CompilerParams(use_tc_tiling_on_sc=False),
    out_type=jax.ShapeDtypeStruct((BATCH, NFIELDS, NEMB), jnp.float32),
    scratch_types=[
        pltpu.VMEM((2, RCHUNK, NFIELDS), jnp.int32),
        pltpu.VMEM((2, RCHUNK, NFIELDS), jnp.float32),
        pltpu.VMEM((2, RCHUNK, NFIELDS, NEMB), jnp.float32),
        pltpu.SemaphoreType.DMA,
        pltpu.SemaphoreType.DMA,
    ],
)
def _emb_lookup(w_hbm, idx_hbm, val_hbm, out_hbm, idx_v, val_v, rows_v, gsem, osem):
    wid = lax.axis_index("s") * NC + lax.axis_index("c")
    row0 = wid * ROWS_PER_W

    def stage(c, buf):
        b0 = row0 + c * RCHUNK
        pltpu.sync_copy(idx_hbm.at[pl.ds(b0, RCHUNK)], idx_v.at[buf])
        pltpu.sync_copy(val_hbm.at[pl.ds(b0, RCHUNK)], val_v.at[buf])

        def fire(r, carry):
            pltpu.async_copy(
                w_hbm.at[idx_v.at[buf, r]], rows_v.at[buf, r], gsem
            )
            return carry

        lax.fori_loop(0, RCHUNK, fire, 0)
        return pltpu.make_async_copy(
            out_hbm.at[pl.ds(b0, RCHUNK)], rows_v.at[buf], gsem
        )

    def scale(buf):
        def row_body(r, carry):
            va = val_v[buf, r, pl.ds(0, NLANE)]
            vb = val_v[buf, r, pl.ds(NFIELDS - NLANE, NLANE)]
            for f in range(NFIELDS):
                s = va[f] if f < NLANE else vb[f - (NFIELDS - NLANE)]
                rows_v[buf, r, f, :] = rows_v[buf, r, f, :] * s
            return carry

        lax.fori_loop(0, RCHUNK, row_body, 0)

    gathers = [None, None]
    out_copies = [None, None]
    gathers[0] = stage(0, 0)
    for c in range(NCHUNK):
        buf = c & 1
        if c + 1 < NCHUNK:
            if out_copies[1 - buf] is not None:
                out_copies[1 - buf].wait()
                out_copies[1 - buf] = None
            gathers[1 - buf] = stage(c + 1, 1 - buf)
        gathers[buf].wait()
        scale(buf)
        out_copies[buf] = pltpu.async_copy(
            rows_v.at[buf], out_hbm.at[pl.ds(row0 + c * RCHUNK, RCHUNK)], osem
        )
    for cp in out_copies:
        if cp is not None:
            cp.wait()


def kernel(id, value, W):
    return _emb_lookup(W, id.astype(jnp.int32), value)
